# baseline (speedup 1.0000x reference)
import functools
import jax
import jax.numpy as jnp
from jax import lax
from jax.experimental import pallas as pl
from jax.experimental.pallas import tpu as pltpu
from jax.experimental.pallas import tpu_sc as plsc

VOCAB = 100000
EMB = 64
REGION = 5
BATCH = 1024
SEQ = 200

NC = 2
NS = 16
NW = NC * NS
ROWS_PER_W = BATCH // NW
LANES = 16
GROUPS = EMB // LANES
UROW = 384
CH = 40
NCH = SEQ // CH
WOFF = 8
WROWS = 216


VSPAN = 384
RALIGNED = 260
RTAIL = VOCAB - RALIGNED * VSPAN
NPLANE = UROW // 128
RUNITS = NPLANE * RALIGNED
RSTEPS = 26


def _relayout_body(u2d_hbm, utail_hbm, u3_hbm, in_v, out_v,
                   sem_i0, sem_i1, sem_o0, sem_o1):
    c = lax.axis_index("c")
    s = lax.axis_index("s")
    wid = s * NC + c
    sem_is = (sem_i0, sem_i1)
    sem_os = (sem_o0, sem_o1)
    lanes = jnp.arange(LANES, dtype=jnp.int32)

    def unit_jv(k):
        uid = wid + NW * k
        return uid // RALIGNED, uid % RALIGNED

    def start_in(k, cg, buf):
        uid = wid + NW * k

        @pl.when(uid < RUNITS)
        def _():
            jj, vb = unit_jv(k)
            roff = pl.multiple_of(128 * jj + 16 * cg, 16)
            voff = pl.multiple_of(vb * VSPAN, 128)
            pltpu.async_copy(
                u2d_hbm.at[pl.ds(roff, 16), pl.ds(voff, VSPAN)],
                in_v.at[buf], sem_is[buf])

    def wait_in(buf):
        pltpu.make_async_copy(
            u2d_hbm.at[pl.ds(0, 16), pl.ds(0, VSPAN)], in_v.at[buf],
            sem_is[buf]).wait()

    def wait_out(p):
        pltpu.make_async_copy(
            out_v.at[p], u3_hbm.at[0, pl.ds(0, VSPAN)], sem_os[p]).wait()

    def do_unit(j, upar):
        k = 2 * j + upar
        uid = wid + NW * k

        @pl.when(uid < RUNITS)
        def _():
            @pl.when(k >= 2)
            def _():
                wait_out(upar)

            for cg in range(8):
                bufin = cg % 2
                if cg < 7:
                    start_in(k, cg + 1, (cg + 1) % 2)
                else:
                    start_in(k + 1, 0, 0)
                wait_in(bufin)

                def vrow(vv, carry):
                    vsplat = jnp.full((LANES,), vv, jnp.int32)
                    val = plsc.load_gather(in_v.at[bufin], [lanes, vsplat])
                    out_v[upar, vv, pl.ds(cg * LANES, LANES)] = val
                    return carry

                lax.fori_loop(0, VSPAN, vrow, 0, unroll=4)

            jj, vb = unit_jv(k)
            voff = pl.multiple_of(vb * VSPAN, 8)
            pltpu.async_copy(
                out_v.at[upar], u3_hbm.at[jj, pl.ds(voff, VSPAN)], sem_os[upar])

    start_in(0, 0, 0)

    def step(j, carry):
        do_unit(j, 0)
        do_unit(j, 1)
        return carry

    lax.fori_loop(0, RSTEPS // 2, step, 0)

    for p in range(2):
        wait_out(p)

    @pl.when(wid == NW - 1)
    def _():
        for jj in range(NPLANE):
            pltpu.sync_copy(utail_hbm.at[jj], out_v.at[0, pl.ds(0, RTAIL)])
            pltpu.sync_copy(
                out_v.at[0, pl.ds(0, RTAIL)],
                u3_hbm.at[jj, pl.ds(RALIGNED * VSPAN, RTAIL)])


def _sc_body(seq_hbm, semb_hbm, u_hbm, out_hbm, idx0_v, idx1_v, rows_v, win_v,
             out_v, sem_seq, sem_win, sem_g0, sem_g1, sem_o0, sem_o1):
    cc = lax.axis_index("c")
    s = lax.axis_index("s")
    wid = s * NC + cc
    row0 = wid * ROWS_PER_W

    zeros = jnp.zeros((LANES,), jnp.float32)
    for q in range(2):
        for prow in (WOFF - 2, WOFF - 1, WOFF + SEQ, WOFF + SEQ + 1):
            for g in range(GROUPS):
                win_v[q, prow, pl.ds(g * LANES, LANES)] = zeros

    idxbufs = (idx0_v, idx1_v)
    sem_gs = (sem_g0, sem_g1)
    sem_os = (sem_o0, sem_o1)

    def start_gather(qidx, ci, p):
        for j in range(NPLANE):
            pltpu.async_copy(
                u_hbm.at[j].at[idxbufs[qidx].at[pl.ds(ci * CH, CH)]],
                rows_v.at[p, j], sem_gs[p])

    def wait_gather(p):
        pltpu.make_async_copy(
            u_hbm.at[pl.ds(0, NPLANE), pl.ds(0, CH)], rows_v.at[p],
            sem_gs[p]).wait()

    def wait_out(p):
        pltpu.make_async_copy(
            out_v.at[p], out_hbm.at[0, pl.ds(0, CH)], sem_os[p]).wait()

    def compute_chunk(q, ci, p, row):
        woff0 = ci * CH + WOFF - 2
        for g in range(GROUPS):
            slg = pl.ds(g * LANES, LANES)
            w0 = win_v[q, woff0, slg]
            w1 = win_v[q, woff0 + 1, slg]
            w2 = win_v[q, woff0 + 2, slg]
            w3 = win_v[q, woff0 + 3, slg]

            def tok(t, carry):
                wa, wb, wc, wd = carry
                we = win_v[q, woff0 + 4 + t, slg]
                ws = (wa, wb, wc, wd, we)
                col0 = g * LANES
                acc = wa * rows_v[p, col0 // 128, t, pl.ds(col0 % 128, LANES)]
                for r in range(1, REGION):
                    colr = r * EMB + g * LANES
                    u = rows_v[p, colr // 128, t, pl.ds(colr % 128, LANES)]
                    acc = jnp.maximum(acc, ws[r] * u)
                out_v[p, t, slg] = acc
                return (wb, wc, wd, we)

            lax.fori_loop(0, CH, tok, (w0, w1, w2, w3), unroll=2)
        pltpu.async_copy(
            out_v.at[p], out_hbm.at[row, pl.ds(ci * CH, CH)], sem_os[p])

    pltpu.sync_copy(seq_hbm.at[row0], idx0_v)
    pltpu.async_copy(semb_hbm.at[row0], win_v.at[0, pl.ds(WOFF, SEQ)], sem_win)
    start_gather(0, 0, 0)

    def step(j, carry):
        for c in range(2):
            row = row0 + 2 * j + c
            nxt = 2 * j + c + 1

            pltpu.make_async_copy(
                semb_hbm.at[row0], win_v.at[c, pl.ds(WOFF, SEQ)], sem_win).wait()

            def prefetch_next():
                pltpu.async_copy(seq_hbm.at[row + 1], idxbufs[1 - c], sem_seq)
                pltpu.async_copy(
                    semb_hbm.at[row + 1], win_v.at[1 - c, pl.ds(WOFF, SEQ)],
                    sem_win)

            if c == 0:
                prefetch_next()
            else:
                @pl.when(j < ROWS_PER_W // 2 - 1)
                def _():
                    prefetch_next()

            for ci in range(NCH):
                p = (c + ci) % 2

                if ci < NCH - 1:
                    start_gather(c, ci + 1, 1 - p)
                else:
                    def next_row_gather():
                        pltpu.make_async_copy(
                            seq_hbm.at[row0], idxbufs[1 - c], sem_seq).wait()
                        start_gather(1 - c, 0, 1 - p)

                    if c == 0:
                        next_row_gather()
                    else:
                        @pl.when(j < ROWS_PER_W // 2 - 1)
                        def _():
                            next_row_gather()

                wait_gather(p)

                if c == 0 and ci < 2:
                    @pl.when(j > 0)
                    def _():
                        wait_out(p)
                else:
                    wait_out(p)

                compute_chunk(c, ci, p, row)
        return carry

    lax.fori_loop(0, ROWS_PER_W // 2, step, 0)
    wait_out(0)
    wait_out(1)


@jax.jit
def _region_embed(seq, seq_emb, U):
    seq2 = seq.astype(jnp.int32)
    mesh = plsc.VectorSubcoreMesh(core_axis_name="c", subcore_axis_name="s")
    u2d = jnp.transpose(U, (1, 2, 0)).reshape(REGION * EMB, VOCAB)
    utail = jnp.pad(
        U[RALIGNED * VSPAN:].reshape(RTAIL, REGION * EMB),
        ((0, 0), (0, UROW - REGION * EMB)))
    utail3 = jnp.transpose(utail.reshape(RTAIL, NPLANE, 128), (1, 0, 2))
    relayout = pl.kernel(
        _relayout_body,
        out_type=jax.ShapeDtypeStruct((NPLANE, VOCAB, 128), jnp.float32),
        mesh=mesh,
        scratch_types=[
            pltpu.VMEM((2, 16, VSPAN), jnp.float32),
            pltpu.VMEM((2, VSPAN, 128), jnp.float32),
            pltpu.SemaphoreType.DMA,
            pltpu.SemaphoreType.DMA,
            pltpu.SemaphoreType.DMA,
            pltpu.SemaphoreType.DMA,
        ],
        compiler_params=pltpu.CompilerParams(needs_layout_passes=False),
    )
    u2 = relayout(u2d, utail3)
    f = pl.kernel(
        _sc_body,
        out_type=jax.ShapeDtypeStruct((BATCH, SEQ, EMB), jnp.float32),
        mesh=mesh,
        scratch_types=[
            pltpu.VMEM((SEQ,), jnp.int32),
            pltpu.VMEM((SEQ,), jnp.int32),
            pltpu.VMEM((2, NPLANE, CH, 128), jnp.float32),
            pltpu.VMEM((2, WROWS, EMB), jnp.float32),
            pltpu.VMEM((2, CH, EMB), jnp.float32),
            pltpu.SemaphoreType.DMA,
            pltpu.SemaphoreType.DMA,
            pltpu.SemaphoreType.DMA,
            pltpu.SemaphoreType.DMA,
            pltpu.SemaphoreType.DMA,
            pltpu.SemaphoreType.DMA,
        ],
    )
    return f(seq2, seq_emb, u2)


def kernel(seq, seq_emb, U):
    return _region_embed(seq, seq_emb, U)


# speedup vs baseline: 1.9098x; 1.9098x over previous
import functools
import jax
import jax.numpy as jnp
from jax import lax
from jax.experimental import pallas as pl
from jax.experimental.pallas import tpu as pltpu
from jax.experimental.pallas import tpu_sc as plsc

VOCAB = 100000
EMB = 64
REGION = 5
BATCH = 1024
SEQ = 200

NC = 2
NS = 16
NW = NC * NS
ROWS_PER_W = BATCH // NW
LANES = 16
GROUPS = EMB // LANES
UROW = 384
CH = 40
NCH = SEQ // CH
WOFF = 8
WROWS = 216


RNB = 40
RCHUNKS = VOCAB // RNB
RSTEPS = 80


def _relayout_body(u_hbm, u2_hbm, in_v, out_v, sem_i0, sem_i1, sem_o0, sem_o1):
    c = lax.axis_index("c")
    s = lax.axis_index("s")
    wid = s * NC + c
    sem_is = (sem_i0, sem_i1)
    sem_os = (sem_o0, sem_o1)

    def start_in(k, buf):
        cid = wid + NW * k

        @pl.when(cid < RCHUNKS)
        def _():
            pltpu.async_copy(
                u_hbm.at[pl.ds(cid * RNB, RNB)], in_v.at[buf], sem_is[buf])

    def do_chunk(k, buf):
        cid = wid + NW * k

        @pl.when(cid < RCHUNKS)
        def _():
            pltpu.make_async_copy(
                u_hbm.at[pl.ds(0, RNB)], in_v.at[buf], sem_is[buf]).wait()

            @pl.when(k >= 2)
            def _():
                pltpu.make_async_copy(
                    out_v.at[buf], u2_hbm.at[pl.ds(0, RNB)], sem_os[buf]).wait()

            def row(t, carry):
                for g in range(REGION * GROUPS):
                    out_v[buf, t, pl.ds(g * LANES, LANES)] = (
                        in_v[buf, t, pl.ds(g * LANES, LANES)])
                return carry

            lax.fori_loop(0, RNB, row, 0, unroll=2)

            pltpu.async_copy(
                out_v.at[buf], u2_hbm.at[pl.ds(cid * RNB, RNB)], sem_os[buf])

    start_in(0, 0)
    start_in(1, 1)

    def step(j, carry):
        k0 = 2 * j
        do_chunk(k0, 0)
        start_in(k0 + 2, 0)
        do_chunk(k0 + 1, 1)
        start_in(k0 + 3, 1)
        return carry

    lax.fori_loop(0, RSTEPS // 2, step, 0)

    for buf in range(2):
        pltpu.make_async_copy(
            out_v.at[buf], u2_hbm.at[pl.ds(0, RNB)], sem_os[buf]).wait()


def _sc_body(seq_hbm, semb_hbm, u_hbm, out_hbm, idx0_v, idx1_v, rows_v, win_v,
             out_v, sem_seq, sem_win, sem_g0, sem_g1, sem_o0, sem_o1):
    cc = lax.axis_index("c")
    s = lax.axis_index("s")
    wid = s * NC + cc
    row0 = wid * ROWS_PER_W

    zeros = jnp.zeros((LANES,), jnp.float32)
    for q in range(2):
        for prow in (WOFF - 2, WOFF - 1, WOFF + SEQ, WOFF + SEQ + 1):
            for g in range(GROUPS):
                win_v[q, prow, pl.ds(g * LANES, LANES)] = zeros

    idxbufs = (idx0_v, idx1_v)
    sem_gs = (sem_g0, sem_g1)
    sem_os = (sem_o0, sem_o1)

    def start_gather(qidx, ci, p):
        pltpu.async_copy(
            u_hbm.at[idxbufs[qidx].at[pl.ds(ci * CH, CH)]], rows_v.at[p],
            sem_gs[p])

    def wait_gather(p):
        pltpu.make_async_copy(
            u_hbm.at[pl.ds(0, CH)], rows_v.at[p], sem_gs[p]).wait()

    def wait_out(p):
        pltpu.make_async_copy(
            out_v.at[p], out_hbm.at[0, pl.ds(0, CH)], sem_os[p]).wait()

    def compute_chunk(q, ci, p, row):
        woff0 = ci * CH + WOFF - 2
        for g in range(GROUPS):
            slg = pl.ds(g * LANES, LANES)
            w0 = win_v[q, woff0, slg]
            w1 = win_v[q, woff0 + 1, slg]
            w2 = win_v[q, woff0 + 2, slg]
            w3 = win_v[q, woff0 + 3, slg]

            def tok(t, carry):
                wa, wb, wc, wd = carry
                we = win_v[q, woff0 + 4 + t, slg]
                acc = wa * rows_v[p, t, pl.ds(g * LANES, LANES)]
                acc = jnp.maximum(acc, wb * rows_v[p, t, pl.ds(EMB + g * LANES, LANES)])
                acc = jnp.maximum(acc, wc * rows_v[p, t, pl.ds(2 * EMB + g * LANES, LANES)])
                acc = jnp.maximum(acc, wd * rows_v[p, t, pl.ds(3 * EMB + g * LANES, LANES)])
                acc = jnp.maximum(acc, we * rows_v[p, t, pl.ds(4 * EMB + g * LANES, LANES)])
                out_v[p, t, slg] = acc
                return (wb, wc, wd, we)

            lax.fori_loop(0, CH, tok, (w0, w1, w2, w3), unroll=2)
        pltpu.async_copy(
            out_v.at[p], out_hbm.at[row, pl.ds(ci * CH, CH)], sem_os[p])

    pltpu.sync_copy(seq_hbm.at[row0], idx0_v)
    pltpu.async_copy(semb_hbm.at[row0], win_v.at[0, pl.ds(WOFF, SEQ)], sem_win)
    start_gather(0, 0, 0)

    def step(j, carry):
        for c in range(2):
            row = row0 + 2 * j + c
            nxt = 2 * j + c + 1

            pltpu.make_async_copy(
                semb_hbm.at[row0], win_v.at[c, pl.ds(WOFF, SEQ)], sem_win).wait()

            def prefetch_next():
                pltpu.async_copy(seq_hbm.at[row + 1], idxbufs[1 - c], sem_seq)
                pltpu.async_copy(
                    semb_hbm.at[row + 1], win_v.at[1 - c, pl.ds(WOFF, SEQ)],
                    sem_win)

            if c == 0:
                prefetch_next()
            else:
                @pl.when(j < ROWS_PER_W // 2 - 1)
                def _():
                    prefetch_next()

            for ci in range(NCH):
                p = (c + ci) % 2

                if ci < NCH - 1:
                    start_gather(c, ci + 1, 1 - p)
                else:
                    def next_row_gather():
                        pltpu.make_async_copy(
                            seq_hbm.at[row0], idxbufs[1 - c], sem_seq).wait()
                        start_gather(1 - c, 0, 1 - p)

                    if c == 0:
                        next_row_gather()
                    else:
                        @pl.when(j < ROWS_PER_W // 2 - 1)
                        def _():
                            next_row_gather()

                wait_gather(p)

                if c == 0 and ci < 2:
                    @pl.when(j > 0)
                    def _():
                        wait_out(p)
                else:
                    wait_out(p)

                compute_chunk(c, ci, p, row)
        return carry

    lax.fori_loop(0, ROWS_PER_W // 2, step, 0)
    wait_out(0)
    wait_out(1)


@jax.jit
def _region_embed(seq, seq_emb, U):
    seq2 = seq.astype(jnp.int32)
    mesh = plsc.VectorSubcoreMesh(core_axis_name="c", subcore_axis_name="s")
    relayout = pl.kernel(
        _relayout_body,
        out_type=jax.ShapeDtypeStruct((VOCAB, UROW), jnp.float32),
        mesh=mesh,
        scratch_types=[
            pltpu.VMEM((2, RNB, REGION * EMB), jnp.float32),
            pltpu.VMEM((2, RNB, UROW), jnp.float32),
            pltpu.SemaphoreType.DMA,
            pltpu.SemaphoreType.DMA,
            pltpu.SemaphoreType.DMA,
            pltpu.SemaphoreType.DMA,
        ],
    )
    u2 = relayout(U.reshape(VOCAB, REGION * EMB))
    f = pl.kernel(
        _sc_body,
        out_type=jax.ShapeDtypeStruct((BATCH, SEQ, EMB), jnp.float32),
        mesh=mesh,
        scratch_types=[
            pltpu.VMEM((SEQ,), jnp.int32),
            pltpu.VMEM((SEQ,), jnp.int32),
            pltpu.VMEM((2, CH, UROW), jnp.float32),
            pltpu.VMEM((2, WROWS, EMB), jnp.float32),
            pltpu.VMEM((2, CH, EMB), jnp.float32),
            pltpu.SemaphoreType.DMA,
            pltpu.SemaphoreType.DMA,
            pltpu.SemaphoreType.DMA,
            pltpu.SemaphoreType.DMA,
            pltpu.SemaphoreType.DMA,
            pltpu.SemaphoreType.DMA,
        ],
    )
    return f(seq2, seq_emb, u2)


def kernel(seq, seq_emb, U):
    return _region_embed(seq, seq_emb, U)


# speedup vs baseline: 1.9878x; 1.0408x over previous
import functools
import jax
import jax.numpy as jnp
from jax import lax
from jax.experimental import pallas as pl
from jax.experimental.pallas import tpu as pltpu
from jax.experimental.pallas import tpu_sc as plsc

VOCAB = 100000
EMB = 64
REGION = 5
BATCH = 1024
SEQ = 200

NC = 2
NS = 16
NW = NC * NS
ROWS_PER_W = BATCH // NW
LANES = 16
GROUPS = EMB // LANES
UROW = 384
CH = 40
NCH = SEQ // CH
WOFF = 8
WROWS = 216


RNB = 40
RCHUNKS = VOCAB // RNB
RSTEPS = 80


def _relayout_body(u_hbm, u2_hbm, in_v, out_v, sem_i0, sem_i1, sem_o0, sem_o1):
    c = lax.axis_index("c")
    s = lax.axis_index("s")
    wid = s * NC + c
    sem_is = (sem_i0, sem_i1)
    sem_os = (sem_o0, sem_o1)

    def start_in(k, buf):
        cid = wid + NW * k

        @pl.when(cid < RCHUNKS)
        def _():
            pltpu.async_copy(
                u_hbm.at[pl.ds(cid * RNB, RNB)], in_v.at[buf], sem_is[buf])

    def do_chunk(k, buf):
        cid = wid + NW * k

        @pl.when(cid < RCHUNKS)
        def _():
            pltpu.make_async_copy(
                u_hbm.at[pl.ds(0, RNB)], in_v.at[buf], sem_is[buf]).wait()

            @pl.when(k >= 2)
            def _():
                pltpu.make_async_copy(
                    out_v.at[buf], u2_hbm.at[pl.ds(0, RNB)], sem_os[buf]).wait()

            def row(t, carry):
                for g in range(REGION * GROUPS):
                    out_v[buf, t, pl.ds(g * LANES, LANES)] = (
                        in_v[buf, t, pl.ds(g * LANES, LANES)])
                return carry

            lax.fori_loop(0, RNB, row, 0, unroll=2)

            pltpu.async_copy(
                out_v.at[buf], u2_hbm.at[pl.ds(cid * RNB, RNB)], sem_os[buf])

    start_in(0, 0)
    start_in(1, 1)

    def step(j, carry):
        k0 = 2 * j
        do_chunk(k0, 0)
        start_in(k0 + 2, 0)
        do_chunk(k0 + 1, 1)
        start_in(k0 + 3, 1)
        return carry

    lax.fori_loop(0, RSTEPS // 2, step, 0)

    for buf in range(2):
        pltpu.make_async_copy(
            out_v.at[buf], u2_hbm.at[pl.ds(0, RNB)], sem_os[buf]).wait()


def _sc_body(seq_hbm, semb_hbm, u_hbm, out_hbm, idx0_v, idx1_v, rows_v, win_v,
             out_v, sem_seq, sem_win, sem_g0, sem_g1, sem_o0, sem_o1):
    cc = lax.axis_index("c")
    s = lax.axis_index("s")
    wid = s * NC + cc
    row0 = wid * ROWS_PER_W

    zeros = jnp.zeros((LANES,), jnp.float32)
    for q in range(2):
        for prow in (WOFF - 2, WOFF - 1, WOFF + SEQ, WOFF + SEQ + 1):
            for g in range(GROUPS):
                win_v[q, prow, pl.ds(g * LANES, LANES)] = zeros

    idxbufs = (idx0_v, idx1_v)
    sem_gs = (sem_g0, sem_g1)
    sem_os = (sem_o0, sem_o1)

    def start_gather(qidx, ci, p):
        pltpu.async_copy(
            u_hbm.at[idxbufs[qidx].at[pl.ds(ci * CH, CH)]], rows_v.at[p],
            sem_gs[p])

    def wait_gather(p):
        pltpu.make_async_copy(
            u_hbm.at[pl.ds(0, CH)], rows_v.at[p], sem_gs[p]).wait()

    def wait_out(p):
        pltpu.make_async_copy(
            out_v.at[p], out_hbm.at[0, pl.ds(0, CH)], sem_os[p]).wait()

    def compute_chunk(q, ci, p, row):
        woff0 = ci * CH + WOFF - 2
        init = []
        for g in range(GROUPS):
            slg = pl.ds(g * LANES, LANES)
            for d in range(4):
                init.append(win_v[q, woff0 + d, slg])

        def tok(t, carry):
            new = []
            for g in range(GROUPS):
                slg = pl.ds(g * LANES, LANES)
                wa, wb, wc, wd = carry[4 * g:4 * g + 4]
                we = win_v[q, woff0 + 4 + t, slg]
                ws = (wa, wb, wc, wd, we)
                acc = wa * rows_v[p, t, pl.ds(g * LANES, LANES)]
                for r in range(1, REGION):
                    u = rows_v[p, t, pl.ds(r * EMB + g * LANES, LANES)]
                    acc = jnp.maximum(acc, ws[r] * u)
                out_v[p, t, slg] = acc
                new.extend([wb, wc, wd, we])
            return tuple(new)

        lax.fori_loop(0, CH, tok, tuple(init), unroll=2)
        pltpu.async_copy(
            out_v.at[p], out_hbm.at[row, pl.ds(ci * CH, CH)], sem_os[p])

    pltpu.sync_copy(seq_hbm.at[row0], idx0_v)
    pltpu.async_copy(semb_hbm.at[row0], win_v.at[0, pl.ds(WOFF, SEQ)], sem_win)
    start_gather(0, 0, 0)

    def step(j, carry):
        for c in range(2):
            row = row0 + 2 * j + c
            nxt = 2 * j + c + 1

            pltpu.make_async_copy(
                semb_hbm.at[row0], win_v.at[c, pl.ds(WOFF, SEQ)], sem_win).wait()

            def prefetch_next():
                pltpu.async_copy(seq_hbm.at[row + 1], idxbufs[1 - c], sem_seq)
                pltpu.async_copy(
                    semb_hbm.at[row + 1], win_v.at[1 - c, pl.ds(WOFF, SEQ)],
                    sem_win)

            if c == 0:
                prefetch_next()
            else:
                @pl.when(j < ROWS_PER_W // 2 - 1)
                def _():
                    prefetch_next()

            for ci in range(NCH):
                p = (c + ci) % 2

                if ci < NCH - 1:
                    start_gather(c, ci + 1, 1 - p)
                else:
                    def next_row_gather():
                        pltpu.make_async_copy(
                            seq_hbm.at[row0], idxbufs[1 - c], sem_seq).wait()
                        start_gather(1 - c, 0, 1 - p)

                    if c == 0:
                        next_row_gather()
                    else:
                        @pl.when(j < ROWS_PER_W // 2 - 1)
                        def _():
                            next_row_gather()

                wait_gather(p)

                if c == 0 and ci < 2:
                    @pl.when(j > 0)
                    def _():
                        wait_out(p)
                else:
                    wait_out(p)

                compute_chunk(c, ci, p, row)
        return carry

    lax.fori_loop(0, ROWS_PER_W // 2, step, 0)
    wait_out(0)
    wait_out(1)


@jax.jit
def _region_embed(seq, seq_emb, U):
    seq2 = seq.astype(jnp.int32)
    mesh = plsc.VectorSubcoreMesh(core_axis_name="c", subcore_axis_name="s")
    relayout = pl.kernel(
        _relayout_body,
        out_type=jax.ShapeDtypeStruct((VOCAB, UROW), jnp.float32),
        mesh=mesh,
        scratch_types=[
            pltpu.VMEM((2, RNB, REGION * EMB), jnp.float32),
            pltpu.VMEM((2, RNB, UROW), jnp.float32),
            pltpu.SemaphoreType.DMA,
            pltpu.SemaphoreType.DMA,
            pltpu.SemaphoreType.DMA,
            pltpu.SemaphoreType.DMA,
        ],
    )
    u2 = relayout(U.reshape(VOCAB, REGION * EMB))
    f = pl.kernel(
        _sc_body,
        out_type=jax.ShapeDtypeStruct((BATCH, SEQ, EMB), jnp.float32),
        mesh=mesh,
        scratch_types=[
            pltpu.VMEM((SEQ,), jnp.int32),
            pltpu.VMEM((SEQ,), jnp.int32),
            pltpu.VMEM((2, CH, UROW), jnp.float32),
            pltpu.VMEM((2, WROWS, EMB), jnp.float32),
            pltpu.VMEM((2, CH, EMB), jnp.float32),
            pltpu.SemaphoreType.DMA,
            pltpu.SemaphoreType.DMA,
            pltpu.SemaphoreType.DMA,
            pltpu.SemaphoreType.DMA,
            pltpu.SemaphoreType.DMA,
            pltpu.SemaphoreType.DMA,
        ],
    )
    return f(seq2, seq_emb, u2)


def kernel(seq, seq_emb, U):
    return _region_embed(seq, seq_emb, U)
